# baseline (speedup 1.0000x reference)
import functools

import jax
import jax.numpy as jnp
from jax import lax
from jax.experimental import pallas as pl
from jax.experimental.pallas import tpu as pltpu
from jax.experimental.pallas import tpu_sc as plsc

N_POS = 64
BOOK = 8192
DIM = 32
BATCH = 16
P = 8

ROWS = BATCH * N_POS
OH_WORDS = ROWS * BOOK
NW = 32
W_WORDS = OH_WORDS // NW
ZCHUNK = 65536

_MESH = functools.partial(
    plsc.VectorSubcoreMesh, core_axis_name="c", subcore_axis_name="s")


def _tc_body(z2_ref, zt_ref, cbt_ref, word_ref, wq_ref, idx_ref):
    g = pl.program_id(0)
    for p in range(P):
        n = g * P + p
        zb = zt_ref[n]
        cbt = cbt_ref[p]
        z2 = z2_ref[n, 0, :].reshape(BATCH, 1)
        zc = jax.lax.dot_general(zb, cbt, (((1,), (0,)), ((), ())))
        c2 = jnp.sum(cbt * cbt, axis=0).reshape(1, BOOK)
        dist = (z2 + c2) - 2.0 * zc
        m = jnp.min(dist, axis=1, keepdims=True)
        iota = jax.lax.broadcasted_iota(jnp.int32, (BATCH, BOOK), 1)
        idx = jnp.min(jnp.where(dist == m, iota, jnp.int32(BOOK)), axis=1)
        one_hot = (iota == idx[:, None]).astype(jnp.float32)
        wq = jax.lax.dot_general(one_hot, cbt, (((1,), (1,)), ((), ())))
        wq_ref[n] = wq
        word_ref[n] = zb + (wq - zb)
        idx_ref[n, 0, :] = idx


@functools.partial(
    pl.kernel,
    out_type=jax.ShapeDtypeStruct((OH_WORDS,), jnp.float32),
    mesh=_MESH(),
    scratch_types=[
        pltpu.VMEM((ZCHUNK,), jnp.float32),
        pltpu.SemaphoreType.DMA,
    ],
)
def _sc_zero(zeros_hbm, out_hbm, zbuf, sem):
    wid = lax.axis_index("s") * 2 + lax.axis_index("c")
    pltpu.sync_copy(zeros_hbm, zbuf)
    base = wid * W_WORDS
    copies = []
    for i in range(W_WORDS // ZCHUNK):
        copies.append(
            pltpu.async_copy(zbuf, out_hbm.at[pl.ds(base + i * ZCHUNK, ZCHUNK)], sem))
    for c in copies:
        c.wait()


@functools.partial(
    pl.kernel,
    mesh=_MESH(),
    scratch_types=[
        pltpu.VMEM((NW,), jnp.int32),
        pltpu.VMEM((NW,), jnp.int32),
        pltpu.VMEM((NW,), jnp.float32),
        pltpu.SemaphoreType.DMA,
    ],
)
def _sc_scatter(idx_hbm, oh_ref, idxv, flatv, onesv, sem):
    wid = lax.axis_index("s") * 2 + lax.axis_index("c")
    base = wid * NW
    pltpu.sync_copy(idx_hbm.at[pl.ds(base, NW)], idxv)
    ones = jnp.ones((16,), jnp.float32)
    for c in range(NW // 16):
        v = idxv[pl.ds(c * 16, 16)]
        row = lax.iota(jnp.int32, 16) + (base + c * 16)
        flatv[pl.ds(c * 16, 16)] = row * jnp.int32(BOOK) + v
        onesv[pl.ds(c * 16, 16)] = ones
    pltpu.async_copy(onesv, oh_ref.at[flatv], sem).wait()


def kernel(z, codebook):
    cbt = jnp.transpose(codebook, (0, 2, 1))
    zt = jnp.transpose(z, (1, 0, 2))
    z2 = jnp.sum(z * z, axis=-1)
    z2t = jnp.transpose(z2, (1, 0)).reshape(N_POS, 1, BATCH)

    oh0 = _sc_zero(jnp.zeros((ZCHUNK,), jnp.float32))

    word_t, wq_t, idx3 = pl.pallas_call(
        _tc_body,
        grid=(N_POS // P,),
        in_specs=[
            pl.BlockSpec((N_POS, 1, BATCH), lambda g: (0, 0, 0)),
            pl.BlockSpec((N_POS, BATCH, DIM), lambda g: (0, 0, 0)),
            pl.BlockSpec((P, DIM, BOOK), lambda g: (g, 0, 0)),
        ],
        out_specs=[
            pl.BlockSpec((N_POS, BATCH, DIM), lambda g: (0, 0, 0)),
            pl.BlockSpec((N_POS, BATCH, DIM), lambda g: (0, 0, 0)),
            pl.BlockSpec((N_POS, 1, BATCH), lambda g: (0, 0, 0)),
        ],
        out_shape=[
            jax.ShapeDtypeStruct((N_POS, BATCH, DIM), jnp.float32),
            jax.ShapeDtypeStruct((N_POS, BATCH, DIM), jnp.float32),
            jax.ShapeDtypeStruct((N_POS, 1, BATCH), jnp.int32),
        ],
        compiler_params=pltpu.CompilerParams(
            dimension_semantics=("arbitrary",),
        ),
    )(z2t, zt, cbt)

    idx = jnp.transpose(idx3.reshape(N_POS, BATCH), (1, 0))

    oh_ref = jax.new_ref(oh0)
    _sc_scatter(idx.reshape(ROWS), oh_ref)
    one_hot = oh_ref[...].reshape(BATCH, N_POS, BOOK)

    word = jnp.transpose(word_t, (1, 0, 2))
    wq = jnp.transpose(wq_t, (1, 0, 2))
    return (word, wq, idx, one_hot)


# speedup vs baseline: 1.4467x; 1.4467x over previous
import jax
import jax.numpy as jnp
from jax.experimental import pallas as pl
from jax.experimental.pallas import tpu as pltpu

N_POS = 64
BOOK = 8192
HALFK = BOOK // 2
DIM = 32
BATCH = 16
P = 8
CHUNKS = N_POS // P
BIG = BOOK


def _tc_body(z2_ref, zt_ref, cbt_ref, word_ref, wq_ref, idx_ref, oh_ref,
             m0s, i0s, wq0s):
    s = pl.program_id(0)
    g = s // 2
    kk = s % 2

    @pl.when(jnp.logical_and(s < 2 * CHUNKS, kk == 0))
    def _phase0():
        for p in range(P):
            n = g * P + p
            zb = zt_ref[n]
            cb0 = cbt_ref[p]
            z2 = z2_ref[n, 0, :].reshape(BATCH, 1)
            zc = jax.lax.dot_general(zb, cb0, (((1,), (0,)), ((), ())))
            c2 = jnp.sum(cb0 * cb0, axis=0).reshape(1, HALFK)
            dist = (z2 + c2) - 2.0 * zc
            m0 = jnp.min(dist, axis=1)
            iota = jax.lax.broadcasted_iota(jnp.int32, (BATCH, HALFK), 1)
            i0 = jnp.min(jnp.where(dist == m0[:, None], iota, BIG), axis=1)
            ohc = (iota == i0[:, None]).astype(jnp.float32)
            wq0 = jax.lax.dot_general(ohc, cb0, (((1,), (1,)), ((), ())))
            m0s[p, 0, :] = m0
            i0s[p, 0, :] = i0
            wq0s[p] = wq0

    @pl.when(jnp.logical_and(kk == 0, s >= 2))
    def _write_prev_half1():
        for p in range(P):
            idxp = idx_ref[(g - 1) * P + p, 0, :]
            iota1 = jax.lax.broadcasted_iota(jnp.int32, (BATCH, HALFK), 1) + HALFK
            oh_ref[:, p, :] = (iota1 == idxp[:, None]).astype(jnp.float32)

    @pl.when(kk == 1)
    def _phase1():
        for p in range(P):
            n = g * P + p
            zb = zt_ref[n]
            cb1 = cbt_ref[p]
            z2 = z2_ref[n, 0, :].reshape(BATCH, 1)
            zc = jax.lax.dot_general(zb, cb1, (((1,), (0,)), ((), ())))
            c2 = jnp.sum(cb1 * cb1, axis=0).reshape(1, HALFK)
            dist = (z2 + c2) - 2.0 * zc
            m1 = jnp.min(dist, axis=1)
            iota0 = jax.lax.broadcasted_iota(jnp.int32, (BATCH, HALFK), 1)
            iota1 = iota0 + HALFK
            i1 = jnp.min(jnp.where(dist == m1[:, None], iota1, BIG), axis=1)
            m0 = m0s[p, 0, :]
            i0 = i0s[p, 0, :]
            mf = jnp.minimum(m0, m1)
            idxf = jnp.minimum(jnp.where(m0 == mf, i0, BIG),
                               jnp.where(m1 == mf, i1, BIG))
            oh_ref[:, p, :] = (iota0 == idxf[:, None]).astype(jnp.float32)
            oh1c = (iota1 == i1[:, None]).astype(jnp.float32)
            wq1 = jax.lax.dot_general(oh1c, cb1, (((1,), (1,)), ((), ())))
            idxf_b = jnp.broadcast_to(idxf[:, None], (BATCH, DIM))
            i0_b = jnp.broadcast_to(i0[:, None], (BATCH, DIM))
            wq = jnp.where(idxf_b == i0_b, wq0s[p], wq1)
            wq_ref[n] = wq
            word_ref[n] = zb + (wq - zb)
            idx_ref[n, 0, :] = idxf


def kernel(z, codebook):
    cbt = jnp.transpose(codebook, (0, 2, 1))
    zt = jnp.transpose(z, (1, 0, 2))
    z2 = jnp.sum(z * z, axis=-1)
    z2t = jnp.transpose(z2, (1, 0)).reshape(N_POS, 1, BATCH)

    nsteps = 2 * CHUNKS + 1

    word_t, wq_t, idx3, one_hot = pl.pallas_call(
        _tc_body,
        grid=(nsteps,),
        in_specs=[
            pl.BlockSpec((N_POS, 1, BATCH), lambda s: (0, 0, 0)),
            pl.BlockSpec((N_POS, BATCH, DIM), lambda s: (0, 0, 0)),
            pl.BlockSpec((P, DIM, HALFK),
                         lambda s: (jnp.minimum(s, 2 * CHUNKS - 1) // 2, 0,
                                    jnp.minimum(s, 2 * CHUNKS - 1) % 2)),
        ],
        out_specs=[
            pl.BlockSpec((N_POS, BATCH, DIM), lambda s: (0, 0, 0)),
            pl.BlockSpec((N_POS, BATCH, DIM), lambda s: (0, 0, 0)),
            pl.BlockSpec((N_POS, 1, BATCH), lambda s: (0, 0, 0)),
            pl.BlockSpec((BATCH, P, HALFK),
                         lambda s: (0, jnp.maximum(s - 1, 0) // 2,
                                    jnp.maximum(s - 1, 0) % 2)),
        ],
        out_shape=[
            jax.ShapeDtypeStruct((N_POS, BATCH, DIM), jnp.float32),
            jax.ShapeDtypeStruct((N_POS, BATCH, DIM), jnp.float32),
            jax.ShapeDtypeStruct((N_POS, 1, BATCH), jnp.int32),
            jax.ShapeDtypeStruct((BATCH, N_POS, BOOK), jnp.float32),
        ],
        scratch_shapes=[
            pltpu.VMEM((P, 1, BATCH), jnp.float32),
            pltpu.VMEM((P, 1, BATCH), jnp.int32),
            pltpu.VMEM((P, BATCH, DIM), jnp.float32),
        ],
        compiler_params=pltpu.CompilerParams(
            dimension_semantics=("arbitrary",),
        ),
    )(z2t, zt, cbt)

    idx = jnp.transpose(idx3.reshape(N_POS, BATCH), (1, 0))
    word = jnp.transpose(word_t, (1, 0, 2))
    wq = jnp.transpose(wq_t, (1, 0, 2))
    return (word, wq, idx, one_hot)


# speedup vs baseline: 2.0856x; 1.4416x over previous
import jax
import jax.numpy as jnp
from jax.experimental import pallas as pl
from jax.experimental.pallas import tpu as pltpu

N_POS = 64
BOOK = 8192
DIM = 32
BATCH = 16
P = 8


def _body(z2_ref, zt_ref, cbt_ref, word_ref, wq_ref, idx_ref, oh_ref):
    g = pl.program_id(0)
    for p in range(P):
        n = g * P + p
        zb = zt_ref[n]
        cbt = cbt_ref[p]
        z2 = z2_ref[n, 0, :].reshape(BATCH, 1)
        zc = jax.lax.dot_general(zb, cbt, (((1,), (0,)), ((), ())))
        c2 = jnp.sum(cbt * cbt, axis=0).reshape(1, BOOK)
        dist = (z2 + c2) - 2.0 * zc
        m = jnp.min(dist, axis=1, keepdims=True)
        iota = jax.lax.broadcasted_iota(jnp.int32, (BATCH, BOOK), 1)
        idx = jnp.min(jnp.where(dist == m, iota, jnp.int32(BOOK)), axis=1)
        one_hot = (iota == idx[:, None]).astype(jnp.float32)
        wq = jax.lax.dot_general(one_hot, cbt, (((1,), (1,)), ((), ())))
        wq_ref[n] = wq
        word_ref[n] = zb + (wq - zb)
        idx_ref[n, 0, :] = idx
        oh_ref[:, p, :] = one_hot


def kernel(z, codebook):
    cbt = jnp.transpose(codebook, (0, 2, 1))
    zt = jnp.transpose(z, (1, 0, 2))
    z2 = jnp.sum(z * z, axis=-1)
    z2t = jnp.transpose(z2, (1, 0)).reshape(N_POS, 1, BATCH)

    word_t, wq_t, idx3, one_hot = pl.pallas_call(
        _body,
        grid=(N_POS // P,),
        in_specs=[
            pl.BlockSpec((N_POS, 1, BATCH), lambda g: (0, 0, 0)),
            pl.BlockSpec((N_POS, BATCH, DIM), lambda g: (0, 0, 0)),
            pl.BlockSpec((P, DIM, BOOK), lambda g: (g, 0, 0)),
        ],
        out_specs=[
            pl.BlockSpec((N_POS, BATCH, DIM), lambda g: (0, 0, 0)),
            pl.BlockSpec((N_POS, BATCH, DIM), lambda g: (0, 0, 0)),
            pl.BlockSpec((N_POS, 1, BATCH), lambda g: (0, 0, 0)),
            pl.BlockSpec((BATCH, P, BOOK), lambda g: (0, g, 0)),
        ],
        out_shape=[
            jax.ShapeDtypeStruct((N_POS, BATCH, DIM), jnp.float32),
            jax.ShapeDtypeStruct((N_POS, BATCH, DIM), jnp.float32),
            jax.ShapeDtypeStruct((N_POS, 1, BATCH), jnp.int32),
            jax.ShapeDtypeStruct((BATCH, N_POS, BOOK), jnp.float32),
        ],
        compiler_params=pltpu.CompilerParams(
            dimension_semantics=("arbitrary",),
        ),
    )(z2t, zt, cbt)

    idx = jnp.transpose(idx3.reshape(N_POS, BATCH), (1, 0))
    word = jnp.transpose(word_t, (1, 0, 2))
    wq = jnp.transpose(wq_t, (1, 0, 2))
    return (word, wq, idx, one_hot)
